# baseline (speedup 1.0000x reference)
import functools

import jax
import jax.numpy as jnp
from jax.experimental import pallas as pl
from jax.experimental.pallas import tpu as pltpu

B, S, H = 4, 4096, 2048
TOP_K = 1024
ROW_BLK = 1024


def _logits_body(x_ref, w_ref, out_ref):
    out_ref[...] = jax.lax.dot_general(
        x_ref[...], w_ref[...],
        dimension_numbers=(((1,), (0,)), ((), ())),
    )


def _logits(x2d, w_pad):
    n_rows = x2d.shape[0]
    grid = (n_rows // ROW_BLK,)
    return pl.pallas_call(
        _logits_body,
        grid=grid,
        in_specs=[
            pl.BlockSpec((ROW_BLK, H), lambda i: (i, 0)),
            pl.BlockSpec((H, 128), lambda i: (0, 0)),
        ],
        out_specs=pl.BlockSpec((ROW_BLK, 128), lambda i: (i, 0)),
        out_shape=jax.ShapeDtypeStruct((n_rows, 128), jnp.float32),
    )(x2d, w_pad)


def kernel(image_features, router_w, router_b):
    x2d = image_features.reshape(B * S, H)
    w_pad = jnp.zeros((H, 128), jnp.float32).at[:, 0].set(router_w[0])
    logits = _logits(x2d, w_pad)[:, 0].reshape(B, S) + router_b[0]
    routing_weights, selected_tokens = jax.lax.top_k(logits, TOP_K)
    routed = jnp.take_along_axis(image_features, selected_tokens[:, :, None], axis=1)
    return routed * routing_weights[:, :, None]
